# baseline (speedup 1.0000x reference)
import jax
import jax.numpy as jnp
from jax.experimental import pallas as pl


_R, _C = 4096, 4096
_BR = 256


def _select_block(idx_ref, x_ref, o_ref):
    o_ref[...] = jnp.where(idx_ref[...] == 1.0, x_ref[...], 0.0)


def kernel(index, x):
    idx2 = index.reshape(_R, _C)
    x2 = x.reshape(_R, _C)
    y = pl.pallas_call(
        _select_block,
        grid=(_R // _BR,),
        in_specs=[
            pl.BlockSpec((_BR, _C), lambda i: (i, 0)),
            pl.BlockSpec((_BR, _C), lambda i: (i, 0)),
        ],
        out_specs=pl.BlockSpec((_BR, _C), lambda i: (i, 0)),
        out_shape=jax.ShapeDtypeStruct((_R, _C), jnp.float32),
    )(idx2, x2)
    return y.reshape(2, -1)


# speedup vs baseline: 7.5606x; 7.5606x over previous
import jax
import jax.numpy as jnp
from jax.experimental import pallas as pl


_N = 8388608
_BC = 524288


def _select_block(idx_ref, x_ref, o_ref):
    o_ref[...] = jnp.where(idx_ref[...] == 1.0, x_ref[...], 0.0)


def kernel(index, x):
    return pl.pallas_call(
        _select_block,
        grid=(_N // _BC,),
        in_specs=[
            pl.BlockSpec((2, _BC), lambda i: (0, i)),
            pl.BlockSpec((2, _BC), lambda i: (0, i)),
        ],
        out_specs=pl.BlockSpec((2, _BC), lambda i: (0, i)),
        out_shape=jax.ShapeDtypeStruct((2, _N), jnp.float32),
    )(index, x)
